# baseline (speedup 1.0000x reference)
import jax
import jax.numpy as jnp
from jax import lax
from jax.experimental import pallas as pl
from jax.experimental.pallas import tpu as pltpu

OBJ_GOAL = 8
OBJ_LAVA = 9
OBJ_AGENT = 10
COL_RED = 0
COL_GREEN = 1
COL_YELLOW = 4

_NBUF = 4


def _body(posj_ref, valj_ref, lay_ref, out_ref, scratch, sems):
    nbuf, bHW = scratch.shape[0], scratch.shape[1]
    blk = scratch.shape[1:]
    i = pl.program_id(0)
    n = pl.num_programs(0)
    s = lax.rem(i, nbuf)

    @pl.when(i >= nbuf)
    def _():
        pltpu.make_async_copy(scratch.at[s], out_ref.at[pl.ds(0, bHW)],
                              sems.at[s]).wait()

    hw_idx = lax.broadcasted_iota(jnp.int32, blk, 0) + i * bHW
    j_odd = lax.broadcasted_iota(jnp.int32, blk, 1) & 1
    posv = posj_ref[...]
    valv = valj_ref[...]
    layb = jnp.broadcast_to(lay_ref[...], blk)
    base = jnp.where(j_odd == 1, 0, layb)
    eq = (hw_idx == posv).astype(jnp.int32)
    scratch[s] = base + eq * (valv - base)
    pltpu.make_async_copy(scratch.at[s], out_ref.at[pl.ds(i * bHW, bHW)],
                          sems.at[s]).start()

    @pl.when(i == n - 1)
    def _():
        for k in range(nbuf):
            pltpu.make_async_copy(scratch.at[k], out_ref.at[pl.ds(0, bHW)],
                                  sems.at[k]).wait()


def kernel(layout, mask_agent):
    B = mask_agent.shape[0]
    H, W = layout.shape[1], layout.shape[2]
    HW = H * W
    NT = B // 128

    lay2d = layout.reshape(H, W).astype(jnp.int32)
    m = mask_agent.astype(jnp.bool_)
    hwgrid = (jnp.arange(H, dtype=jnp.int32)[:, None] * W
              + jnp.arange(W, dtype=jnp.int32)[None, :])
    pos = jnp.sum(jnp.where(m, hwgrid[None], 0), axis=(1, 2))
    lval = jnp.sum(jnp.where(m, lay2d[None], 0), axis=(1, 2))
    color = jnp.where(lval == OBJ_LAVA, COL_YELLOW,
                      jnp.where(lval == OBJ_GOAL, COL_GREEN, COL_RED))

    j_odd = (jnp.arange(2 * NT, dtype=jnp.int32) & 1)[:, None]
    pos_t = pos.reshape(NT, 1, 128)
    posj = jnp.broadcast_to(pos_t, (NT, 2, 128)).reshape(1, 2 * NT, 128)
    col_t = color.reshape(NT, 1, 128)
    colj = jnp.broadcast_to(col_t, (NT, 2, 128)).reshape(2 * NT, 128)
    valj = jnp.where(j_odd == 1, colj, OBJ_AGENT).reshape(1, 2 * NT, 128)

    lay_r = jnp.broadcast_to(lay2d.reshape(HW, 1, 1), (HW, 1, 128))

    bHW = 128
    out5 = pl.pallas_call(
        _body,
        grid=(HW // bHW,),
        in_specs=[
            pl.BlockSpec((1, 2 * NT, 128), lambda i: (0, 0, 0)),
            pl.BlockSpec((1, 2 * NT, 128), lambda i: (0, 0, 0)),
            pl.BlockSpec((bHW, 1, 128), lambda i: (i, 0, 0)),
        ],
        out_specs=pl.BlockSpec(memory_space=pl.ANY),
        out_shape=jax.ShapeDtypeStruct((HW, 2 * NT, 128), jnp.int32),
        scratch_shapes=[
            pltpu.VMEM((_NBUF, bHW, 2 * NT, 128), jnp.int32),
            pltpu.SemaphoreType.DMA((_NBUF,)),
        ],
    )(posj, valj, lay_r)

    out = out5.reshape(H, W, NT, 2, 128).transpose(2, 4, 0, 1, 3)
    return out.reshape(B, H, W, 2)
